# baseline (speedup 1.0000x reference)
import jax
import jax.numpy as jnp
from jax.experimental import pallas as pl
from jax.experimental.pallas import tpu as pltpu

N = 128
M = 128
IB = 32

_BF = jnp.bfloat16
_F32 = jnp.float32


def _mm3(x, w3):
    xh = x.astype(_BF)
    xl = (x - xh.astype(_F32)).astype(_BF)
    lhs = jnp.concatenate([xh, xh, xl], axis=1)
    return jnp.dot(lhs, w3, preferred_element_type=_F32)


def _mm_big(x, w):
    return jnp.dot(x, w, preferred_element_type=_F32)


def _edge_conv(x, e_aug, wxd3, wd3, ba, wb, bb, we64):
    a = _mm3(x, wxd3) + ba
    bj = _mm3(x, wd3)
    outs = []
    for t in range(N // IB):
        sl = slice(t * IB, (t + 1) * IB)
        l1 = a[sl][:, None, :] + bj[None, :, :]
        if we64 is not None:
            eb = e_aug[sl, :, 0:32].reshape(IB * N, 32)
            eb64 = jnp.concatenate([eb, eb], axis=1)
            l1 = l1 + _mm_big(eb64, we64).reshape(IB, N, M)
        l1 = jnp.maximum(l1, 0.0)
        z = _mm_big(l1.reshape(IB * N, M), wb).reshape(IB, N, M)
        z = z + e_aug[sl, :, 32:33].astype(_F32)
        red = jnp.max(z, axis=1)
        outs.append(jnp.maximum(red + bb, 0.0))
    return jnp.concatenate(outs, axis=0)


def _body(coord_ref, node_ref, edge_ref, joint_ref,
          w1, b1, w2, b2,
          e1_xd, e1_d, e1_we, e1_ba, e1_wb, e1_bb,
          e2_xd, e2_d, e2_we, e2_ba, e2_wb, e2_bb,
          w3a, w3b, b3,
          c1_xd, c1_d, c1_ba, c1_wb, c1_bb,
          c2_xd, c2_d, c2_ba, c2_wb, c2_bb,
          c3_xd, c3_d, c3_ba, c3_wb, c3_bb,
          wo1, bo1, wo2, bo2,
          y1_ref, y2_ref):
    coord = coord_ref[0]
    node = node_ref[0]
    e_aug = edge_ref[0]
    joint = joint_ref[0]

    x = jnp.maximum(_mm3(coord, w1[...]) + b1[...], 0.0)
    x = jnp.maximum(_mm3(x, w2[...]) + b2[...], 0.0)
    x = jnp.concatenate([x, node, joint], axis=1)

    x = _edge_conv(x, e_aug, e1_xd[...], e1_d[...], e1_ba[...],
                   e1_wb[...], e1_bb[...], e1_we[...])
    x = _edge_conv(x, e_aug, e2_xd[...], e2_d[...], e2_ba[...],
                   e2_wb[...], e2_bb[...], e2_we[...])

    g = jnp.max(x, axis=0, keepdims=True)
    x = jnp.maximum(_mm3(x, w3a[...]) + _mm3(g, w3b[...]) + b3[...], 0.0)

    x = _edge_conv(x, e_aug, c1_xd[...], c1_d[...], c1_ba[...],
                   c1_wb[...], c1_bb[...], None)
    ec1 = x
    x = _edge_conv(x, e_aug, c2_xd[...], c2_d[...], c2_ba[...],
                   c2_wb[...], c2_bb[...], None)
    ec2 = x
    x = x + ec1
    x = _edge_conv(x, e_aug, c3_xd[...], c3_d[...], c3_ba[...],
                   c3_wb[...], c3_bb[...], None)
    x = x + ec2

    wo1v, bo1v = wo1[...], bo1[...]
    y1 = jnp.maximum(_mm3(x, wo1v) + bo1v, 0.0)
    y1 = jnp.maximum(_mm3(y1, wo1v) + bo1v, 0.0)
    wo2v, bo2v = wo2[...], bo2[...]
    y2 = jnp.maximum(_mm3(x, wo2v) + bo2v, 0.0)
    y2 = jnp.maximum(_mm3(y2, wo2v) + bo2v, 0.0)
    y1_ref[0] = y1
    y2_ref[0] = y2


def _split3(W):
    Wh = W.astype(_BF)
    Wl = (W - Wh.astype(_F32)).astype(_BF)
    return jnp.concatenate([Wh, Wl, Wh], axis=0)


def kernel(coordinates, adjacency, node_features, edge_features, joint_types, params):
    B = coordinates.shape[0]

    coords = jnp.pad(coordinates, ((0, 0), (0, 0), (0, 8 - coordinates.shape[-1])))
    madd = jnp.where(adjacency > 0, 0.0, -1e9).astype(_F32)
    e_hi = edge_features.astype(_BF)
    e_lo = edge_features - e_hi.astype(_F32)
    e_aug = jnp.concatenate(
        [e_hi.astype(_F32), e_lo, madd[..., None]], axis=-1).astype(_BF)

    def bias(b):
        return b.reshape(1, -1)

    def conv_weights(name_a, name_b, d, with_e):
        Wa, ba = params[name_a]
        Wb, bb = params[name_b]
        wxd3 = _split3(Wa[0:d] - Wa[d:2 * d])
        wd3 = _split3(Wa[d:2 * d])
        ops = [wxd3, wd3]
        if with_e:
            we = Wa[2 * d:]
            weh = we.astype(_BF)
            wel = (we - weh.astype(_F32)).astype(_BF)
            ops.append(jnp.concatenate([weh, weh, wel, wel], axis=0))
        ops += [bias(ba), Wb, bias(bb)]
        return ops

    w1, b1 = params['h1']
    w2, b2 = params['h2']
    w3, b3 = params['h3']
    wo1, bo1 = params['out1']
    wo2, bo2 = params['out2']

    weight_list = [_split3(jnp.pad(w1, ((0, 5), (0, 0)))), bias(b1),
                   _split3(w2), bias(b2)]
    weight_list += conv_weights('ece1_a', 'ece1_b', 104, True)
    weight_list += conv_weights('ece2_a', 'ece2_b', M, True)
    weight_list += [_split3(w3[0:M]), _split3(w3[M:2 * M]), bias(b3)]
    weight_list += conv_weights('ec1_a', 'ec1_b', M, False)
    weight_list += conv_weights('ec2_a', 'ec2_b', M, False)
    weight_list += conv_weights('ec3_a', 'ec3_b', M, False)
    weight_list += [_split3(wo1), bias(bo1), _split3(wo2), bias(bo2)]

    data = [coords, node_features, e_aug, joint_types]

    def data_spec(arr):
        blk = (1,) + arr.shape[1:]
        nd = len(blk)
        return pl.BlockSpec(blk, lambda b, _nd=nd: (b,) + (0,) * (_nd - 1))

    def w_spec(arr):
        nd = arr.ndim
        return pl.BlockSpec(arr.shape, lambda b, _nd=nd: (0,) * _nd)

    in_specs = [data_spec(a) for a in data] + [w_spec(w) for w in weight_list]
    out_specs = [pl.BlockSpec((1, N, M), lambda b: (b, 0, 0)),
                 pl.BlockSpec((1, N, M), lambda b: (b, 0, 0))]
    out_shape = [jax.ShapeDtypeStruct((B, N, M), _F32),
                 jax.ShapeDtypeStruct((B, N, M), _F32)]

    y1, y2 = pl.pallas_call(
        _body,
        grid=(B,),
        in_specs=in_specs,
        out_specs=out_specs,
        out_shape=out_shape,
        compiler_params=pltpu.CompilerParams(
            dimension_semantics=("parallel",)),
    )(*data, *weight_list)
    return (y1, y2)


# speedup vs baseline: 1.3703x; 1.3703x over previous
import jax
import jax.numpy as jnp
from jax.experimental import pallas as pl

N = 128
M = 128
IB = 32

_BF = jnp.bfloat16
_F32 = jnp.float32


def _mm3(x, w3):
    xh = x.astype(_BF)
    xl = (x - xh.astype(_F32)).astype(_BF)
    lhs = jnp.concatenate([xh, xh, xl], axis=1)
    return jnp.dot(lhs, w3, preferred_element_type=_F32)


def _mm_big(x, w):
    return jnp.dot(x, w, preferred_element_type=_F32)


def _edge_conv(x, e_aug, wxd3, wd3, ba, wb, bb, we48):
    a = _mm3(x, wxd3) + ba
    bj = _mm3(x, wd3)
    outs = []
    for t in range(N // IB):
        sl = slice(t * IB, (t + 1) * IB)
        l1 = a[sl][:, None, :] + bj[None, :, :]
        if we48 is not None:
            eb = e_aug[sl, :, 0:16].reshape(IB * N, 16)
            l1 = l1 + _mm3(eb, we48).reshape(IB, N, M)
        l1 = jnp.maximum(l1, 0.0)
        z = _mm_big(l1.reshape(IB * N, M), wb).reshape(IB, N, M)
        z = z + e_aug[sl, :, 16:17]
        red = jnp.max(z, axis=1)
        outs.append(jnp.maximum(red + bb, 0.0))
    return jnp.concatenate(outs, axis=0)


def _body(coord_ref, node_ref, edge_ref, joint_ref,
          w1, b1, w2, b2,
          e1_xd, e1_d, e1_we, e1_ba, e1_wb, e1_bb,
          e2_xd, e2_d, e2_we, e2_ba, e2_wb, e2_bb,
          w3a, w3b, b3,
          c1_xd, c1_d, c1_ba, c1_wb, c1_bb,
          c2_xd, c2_d, c2_ba, c2_wb, c2_bb,
          c3_xd, c3_d, c3_ba, c3_wb, c3_bb,
          wo1, bo1, wo2, bo2,
          y1_ref, y2_ref):
    coord = coord_ref[0]
    node = node_ref[0]
    e_aug = edge_ref[0]
    joint = joint_ref[0]

    x = jnp.maximum(_mm3(coord, w1[...]) + b1[...], 0.0)
    x = jnp.maximum(_mm3(x, w2[...]) + b2[...], 0.0)
    x = jnp.concatenate([x, node, joint], axis=1)

    x = _edge_conv(x, e_aug, e1_xd[...], e1_d[...], e1_ba[...],
                   e1_wb[...], e1_bb[...], e1_we[...])
    x = _edge_conv(x, e_aug, e2_xd[...], e2_d[...], e2_ba[...],
                   e2_wb[...], e2_bb[...], e2_we[...])

    g = jnp.max(x, axis=0, keepdims=True)
    x = jnp.maximum(_mm3(x, w3a[...]) + _mm3(g, w3b[...]) + b3[...], 0.0)

    x = _edge_conv(x, e_aug, c1_xd[...], c1_d[...], c1_ba[...],
                   c1_wb[...], c1_bb[...], None)
    ec1 = x
    x = _edge_conv(x, e_aug, c2_xd[...], c2_d[...], c2_ba[...],
                   c2_wb[...], c2_bb[...], None)
    ec2 = x
    x = x + ec1
    x = _edge_conv(x, e_aug, c3_xd[...], c3_d[...], c3_ba[...],
                   c3_wb[...], c3_bb[...], None)
    x = x + ec2

    wo1v, bo1v = wo1[...], bo1[...]
    y1 = jnp.maximum(_mm3(x, wo1v) + bo1v, 0.0)
    y1 = jnp.maximum(_mm3(y1, wo1v) + bo1v, 0.0)
    wo2v, bo2v = wo2[...], bo2[...]
    y2 = jnp.maximum(_mm3(x, wo2v) + bo2v, 0.0)
    y2 = jnp.maximum(_mm3(y2, wo2v) + bo2v, 0.0)
    y1_ref[0] = y1
    y2_ref[0] = y2


def _split3(W):
    Wh = W.astype(_BF)
    Wl = (W - Wh.astype(_F32)).astype(_BF)
    return jnp.concatenate([Wh, Wl, Wh], axis=0)


def kernel(coordinates, adjacency, node_features, edge_features, joint_types, params):
    B = coordinates.shape[0]

    coords = jnp.pad(coordinates, ((0, 0), (0, 0), (0, 8 - coordinates.shape[-1])))
    madd = jnp.where(adjacency > 0, 0.0, -1e9).astype(_F32)
    e_aug = jnp.concatenate(
        [edge_features, madd[..., None]], axis=-1)

    def bias(b):
        return b.reshape(1, -1)

    def conv_weights(name_a, name_b, d, with_e):
        Wa, ba = params[name_a]
        Wb, bb = params[name_b]
        wxd3 = _split3(Wa[0:d] - Wa[d:2 * d])
        wd3 = _split3(Wa[d:2 * d])
        ops = [wxd3, wd3]
        if with_e:
            ops.append(_split3(Wa[2 * d:]))
        ops += [bias(ba), Wb, bias(bb)]
        return ops

    w1, b1 = params['h1']
    w2, b2 = params['h2']
    w3, b3 = params['h3']
    wo1, bo1 = params['out1']
    wo2, bo2 = params['out2']

    weight_list = [_split3(jnp.pad(w1, ((0, 5), (0, 0)))), bias(b1),
                   _split3(w2), bias(b2)]
    weight_list += conv_weights('ece1_a', 'ece1_b', 104, True)
    weight_list += conv_weights('ece2_a', 'ece2_b', M, True)
    weight_list += [_split3(w3[0:M]), _split3(w3[M:2 * M]), bias(b3)]
    weight_list += conv_weights('ec1_a', 'ec1_b', M, False)
    weight_list += conv_weights('ec2_a', 'ec2_b', M, False)
    weight_list += conv_weights('ec3_a', 'ec3_b', M, False)
    weight_list += [_split3(wo1), bias(bo1), _split3(wo2), bias(bo2)]

    data = [coords, node_features, e_aug, joint_types]

    def data_spec(arr):
        blk = (1,) + arr.shape[1:]
        nd = len(blk)
        return pl.BlockSpec(blk, lambda b, _nd=nd: (b,) + (0,) * (_nd - 1))

    def w_spec(arr):
        nd = arr.ndim
        return pl.BlockSpec(arr.shape, lambda b, _nd=nd: (0,) * _nd)

    in_specs = [data_spec(a) for a in data] + [w_spec(w) for w in weight_list]
    out_specs = [pl.BlockSpec((1, N, M), lambda b: (b, 0, 0)),
                 pl.BlockSpec((1, N, M), lambda b: (b, 0, 0))]
    out_shape = [jax.ShapeDtypeStruct((B, N, M), _F32),
                 jax.ShapeDtypeStruct((B, N, M), _F32)]

    y1, y2 = pl.pallas_call(
        _body,
        grid=(B,),
        in_specs=in_specs,
        out_specs=out_specs,
        out_shape=out_shape,
    )(*data, *weight_list)
    return (y1, y2)


# speedup vs baseline: 1.3720x; 1.0013x over previous
import jax
import jax.numpy as jnp
from jax.experimental import pallas as pl
from jax.experimental.pallas import tpu as pltpu

N = 128
M = 128
IB = 32

_BF = jnp.bfloat16
_F32 = jnp.float32


def _mm3(x, w3):
    xh = x.astype(_BF)
    xl = (x - xh.astype(_F32)).astype(_BF)
    lhs = jnp.concatenate([xh, xh, xl], axis=1)
    return jnp.dot(lhs, w3, preferred_element_type=_F32)


def _mm_big(x, w):
    return jnp.dot(x, w, preferred_element_type=_F32)


def _edge_conv(x, e_aug, wxd3, wd3, ba, wb, bb, we48):
    a = _mm3(x, wxd3) + ba
    bj = _mm3(x, wd3)
    outs = []
    for t in range(N // IB):
        sl = slice(t * IB, (t + 1) * IB)
        l1 = a[sl][:, None, :] + bj[None, :, :]
        if we48 is not None:
            eb = e_aug[sl, :, 0:16].reshape(IB * N, 16)
            l1 = l1 + _mm3(eb, we48).reshape(IB, N, M)
        l1 = jnp.maximum(l1, 0.0)
        z = _mm_big(l1.reshape(IB * N, M), wb).reshape(IB, N, M)
        z = z + e_aug[sl, :, 16:17]
        red = jnp.max(z, axis=1)
        outs.append(jnp.maximum(red + bb, 0.0))
    return jnp.concatenate(outs, axis=0)


def _body(coord_ref, node_ref, edge_ref, joint_ref,
          w1, b1, w2, b2,
          e1_xd, e1_d, e1_we, e1_ba, e1_wb, e1_bb,
          e2_xd, e2_d, e2_we, e2_ba, e2_wb, e2_bb,
          w3a, w3b, b3,
          c1_xd, c1_d, c1_ba, c1_wb, c1_bb,
          c2_xd, c2_d, c2_ba, c2_wb, c2_bb,
          c3_xd, c3_d, c3_ba, c3_wb, c3_bb,
          wo1, bo1, wo2, bo2,
          y1_ref, y2_ref):
    coord = coord_ref[0]
    node = node_ref[0]
    e_aug = edge_ref[0]
    joint = joint_ref[0]

    x = jnp.maximum(_mm3(coord, w1[...]) + b1[...], 0.0)
    x = jnp.maximum(_mm3(x, w2[...]) + b2[...], 0.0)
    x = jnp.concatenate([x, node, joint], axis=1)

    x = _edge_conv(x, e_aug, e1_xd[...], e1_d[...], e1_ba[...],
                   e1_wb[...], e1_bb[...], e1_we[...])
    x = _edge_conv(x, e_aug, e2_xd[...], e2_d[...], e2_ba[...],
                   e2_wb[...], e2_bb[...], e2_we[...])

    g = jnp.max(x, axis=0, keepdims=True)
    x = jnp.maximum(_mm3(x, w3a[...]) + _mm3(g, w3b[...]) + b3[...], 0.0)

    x = _edge_conv(x, e_aug, c1_xd[...], c1_d[...], c1_ba[...],
                   c1_wb[...], c1_bb[...], None)
    ec1 = x
    x = _edge_conv(x, e_aug, c2_xd[...], c2_d[...], c2_ba[...],
                   c2_wb[...], c2_bb[...], None)
    ec2 = x
    x = x + ec1
    x = _edge_conv(x, e_aug, c3_xd[...], c3_d[...], c3_ba[...],
                   c3_wb[...], c3_bb[...], None)
    x = x + ec2

    wo1v, bo1v = wo1[...], bo1[...]
    y1 = jnp.maximum(_mm3(x, wo1v) + bo1v, 0.0)
    y1 = jnp.maximum(_mm3(y1, wo1v) + bo1v, 0.0)
    wo2v, bo2v = wo2[...], bo2[...]
    y2 = jnp.maximum(_mm3(x, wo2v) + bo2v, 0.0)
    y2 = jnp.maximum(_mm3(y2, wo2v) + bo2v, 0.0)
    y1_ref[0] = y1
    y2_ref[0] = y2


def _split3(W):
    Wh = W.astype(_BF)
    Wl = (W - Wh.astype(_F32)).astype(_BF)
    return jnp.concatenate([Wh, Wl, Wh], axis=0)


def kernel(coordinates, adjacency, node_features, edge_features, joint_types, params):
    B = coordinates.shape[0]

    coords = jnp.pad(coordinates, ((0, 0), (0, 0), (0, 8 - coordinates.shape[-1])))
    madd = jnp.where(adjacency > 0, 0.0, -1e9).astype(_F32)
    e_aug = jnp.concatenate(
        [edge_features, madd[..., None]], axis=-1)

    def bias(b):
        return b.reshape(1, -1)

    def conv_weights(name_a, name_b, d, with_e):
        Wa, ba = params[name_a]
        Wb, bb = params[name_b]
        wxd3 = _split3(Wa[0:d] - Wa[d:2 * d])
        wd3 = _split3(Wa[d:2 * d])
        ops = [wxd3, wd3]
        if with_e:
            ops.append(_split3(Wa[2 * d:]))
        ops += [bias(ba), Wb, bias(bb)]
        return ops

    w1, b1 = params['h1']
    w2, b2 = params['h2']
    w3, b3 = params['h3']
    wo1, bo1 = params['out1']
    wo2, bo2 = params['out2']

    weight_list = [_split3(jnp.pad(w1, ((0, 5), (0, 0)))), bias(b1),
                   _split3(w2), bias(b2)]
    weight_list += conv_weights('ece1_a', 'ece1_b', 104, True)
    weight_list += conv_weights('ece2_a', 'ece2_b', M, True)
    weight_list += [_split3(w3[0:M]), _split3(w3[M:2 * M]), bias(b3)]
    weight_list += conv_weights('ec1_a', 'ec1_b', M, False)
    weight_list += conv_weights('ec2_a', 'ec2_b', M, False)
    weight_list += conv_weights('ec3_a', 'ec3_b', M, False)
    weight_list += [_split3(wo1), bias(bo1), _split3(wo2), bias(bo2)]

    data = [coords, node_features, e_aug, joint_types]

    def data_spec(arr):
        blk = (1,) + arr.shape[1:]
        nd = len(blk)
        return pl.BlockSpec(blk, lambda b, _nd=nd: (b,) + (0,) * (_nd - 1))

    def w_spec(arr):
        nd = arr.ndim
        return pl.BlockSpec(arr.shape, lambda b, _nd=nd: (0,) * _nd)

    in_specs = [data_spec(a) for a in data] + [w_spec(w) for w in weight_list]
    out_specs = [pl.BlockSpec((1, N, M), lambda b: (b, 0, 0)),
                 pl.BlockSpec((1, N, M), lambda b: (b, 0, 0))]
    out_shape = [jax.ShapeDtypeStruct((B, N, M), _F32),
                 jax.ShapeDtypeStruct((B, N, M), _F32)]

    y1, y2 = pl.pallas_call(
        _body,
        grid=(B,),
        in_specs=in_specs,
        out_specs=out_specs,
        out_shape=out_shape,
        compiler_params=pltpu.CompilerParams(
            dimension_semantics=("parallel",)),
    )(*data, *weight_list)
    return (y1, y2)
